# Initial kernel scaffold; baseline (speedup 1.0000x reference)
import jax
import jax.numpy as jnp
from jax.experimental import pallas as pl


def kernel(inputs, neighbors, table, W1, f1w1, f1b1, f2w1, f2b1, bout1, W2, f1w2, f1b2, f2w2, f2b2, bout2):
    raise NotImplementedError("write your pallas kernel here")


# speedup vs baseline: 1.2207x; 1.2207x over previous
import functools

import jax
import jax.numpy as jnp
from jax import lax
from jax.experimental import pallas as pl
from jax.experimental.pallas import tpu as pltpu
from jax.experimental.pallas import tpu_sc as plsc

BATCH = 2048
NB1 = 32
FEAT = 256
HID = 256
OUT = 128
HEADS = 2
BB = 8
ROWS = BB * NB1
GW = 128


def _leaky(x):
    return jnp.where(x >= 0, x, 0.2 * x)


def _gather(table, idx):
    n = idx.shape[1]
    mesh = plsc.VectorSubcoreMesh(core_axis_name="core", subcore_axis_name="subcore")

    @functools.partial(
        pl.kernel,
        out_type=jax.ShapeDtypeStruct((n, FEAT), jnp.float32),
        mesh=mesh,
    )
    def gk(table_hbm, idx_hbm, out_hbm):
        def body(i_vmem, o_vmem):
            pltpu.sync_copy(table_hbm.at[i_vmem.at[0]], o_vmem)

        pltpu.emit_pipeline(
            body,
            grid=(n // GW,),
            in_specs=[pl.BlockSpec((1, GW), index_map=lambda i: (0, i))],
            out_specs=[pl.BlockSpec((GW, FEAT), index_map=lambda i: (i, 0))],
            core_axis_name=("core", "subcore"),
            dimension_semantics=(pltpu.PARALLEL,),
        )(idx_hbm, out_hbm)

    return gk(table, idx)


def _att_body(seq_ref, W1_ref, f1w1_ref, f1b1_ref, f2w1_ref, f2b1_ref, bout1_ref,
              W2_ref, f1w2_ref, f1b2_ref, f2w2_ref, f2b2_ref, bout2_ref, out_ref):
    X = seq_ref[...]
    rid = lax.broadcasted_iota(jnp.int32, (ROWS, ROWS), 0) // NB1
    cid = lax.broadcasted_iota(jnp.int32, (ROWS, ROWS), 1) // NB1
    mask = rid == cid

    vals = []
    for h in range(HEADS):
        S = jnp.dot(X, W1_ref[h], preferred_element_type=jnp.float32)
        f1 = jnp.dot(S, f1w1_ref[h], preferred_element_type=jnp.float32) + f1b1_ref[h]
        f2 = jnp.dot(S, f2w1_ref[h], preferred_element_type=jnp.float32) + f2b1_ref[h]
        Z = _leaky(f1 + f2.reshape(1, ROWS))
        Z = jnp.where(mask, Z, -1e30)
        A = jax.nn.softmax(Z, axis=-1)
        V = jnp.dot(A, S, preferred_element_type=jnp.float32) + bout1_ref[h]
        vals.append(V)
    h1 = jnp.concatenate(vals, axis=-1)

    rid8 = lax.broadcasted_iota(jnp.int32, (BB, ROWS), 0)
    cid8 = lax.broadcasted_iota(jnp.int32, (BB, ROWS), 1) // NB1
    mask8 = rid8 == cid8
    acc = jnp.zeros((BB, OUT), jnp.float32)
    for h in range(HEADS):
        S2 = jnp.dot(h1, W2_ref[h], preferred_element_type=jnp.float32)
        f1 = jnp.dot(S2, f1w2_ref[h], preferred_element_type=jnp.float32) + f1b2_ref[h]
        f2 = jnp.dot(S2, f2w2_ref[h], preferred_element_type=jnp.float32) + f2b2_ref[h]
        f1_sel = f1.reshape(BB, NB1)[:, 0:1]
        Z2 = _leaky(f1_sel + f2.reshape(1, ROWS))
        Z2 = jnp.where(mask8, Z2, -1e30)
        A2 = jax.nn.softmax(Z2, axis=-1)
        V2 = jnp.dot(A2, S2, preferred_element_type=jnp.float32) + bout2_ref[h]
        acc = acc + V2
    out_ref[...] = acc * (1.0 / HEADS)


def _attention(seq, W1, f1w1, f1b1, f2w1, f2b1, bout1,
               W2, f1w2, f1b2, f2w2, f2b2, bout2):
    grid = BATCH // BB
    full = lambda *s: pl.BlockSpec(s, lambda i: tuple(0 for _ in s))
    return pl.pallas_call(
        _att_body,
        grid=(grid,),
        in_specs=[
            pl.BlockSpec((ROWS, FEAT), lambda i: (i, 0)),
            full(HEADS, FEAT, HID),
            full(HEADS, HID, 1),
            full(HEADS, 1),
            full(HEADS, HID, 1),
            full(HEADS, 1),
            full(HEADS, HID),
            full(HEADS, HEADS * HID, OUT),
            full(HEADS, OUT, 1),
            full(HEADS, 1),
            full(HEADS, OUT, 1),
            full(HEADS, 1),
            full(HEADS, OUT),
        ],
        out_specs=pl.BlockSpec((BB, OUT), lambda i: (i, 0)),
        out_shape=jax.ShapeDtypeStruct((BATCH, OUT), jnp.float32),
    )(seq, W1, f1w1, f1b1, f2w1, f2b1, bout1, W2, f1w2, f1b2, f2w2, f2b2, bout2)


def kernel(inputs, neighbors, table, W1, f1w1, f1b1, f2w1, f2b1, bout1,
           W2, f1w2, f1b2, f2w2, f2b2, bout2):
    idx = jnp.concatenate([inputs[:, None], neighbors], axis=1)
    idx = idx.reshape(1, BATCH * NB1).astype(jnp.int32)
    seq = _gather(table, idx)
    return _attention(seq, W1, f1w1, f1b1, f2w1, f2b1, bout1,
                      W2, f1w2, f1b2, f2w2, f2b2, bout2)


# speedup vs baseline: 1.2567x; 1.0295x over previous
import functools

import jax
import jax.numpy as jnp
from jax import lax
from jax.experimental import pallas as pl
from jax.experimental.pallas import tpu as pltpu
from jax.experimental.pallas import tpu_sc as plsc

BATCH = 2048
NB1 = 32
FEAT = 256
HID = 256
OUT = 128
HEADS = 2
BB = 8
ROWS = BB * NB1
GW = 128


def _leaky(x):
    return jnp.where(x >= 0, x, 0.2 * x)


def _gather(table, idx):
    n = idx.shape[1]
    mesh = plsc.VectorSubcoreMesh(core_axis_name="core", subcore_axis_name="subcore")

    @functools.partial(
        pl.kernel,
        out_type=jax.ShapeDtypeStruct((n, FEAT), jnp.float32),
        mesh=mesh,
    )
    def gk(table_hbm, idx_hbm, out_hbm):
        def body(i_vmem, o_vmem):
            pltpu.sync_copy(table_hbm.at[i_vmem.at[0]], o_vmem)

        pltpu.emit_pipeline(
            body,
            grid=(n // GW,),
            in_specs=[pl.BlockSpec((1, GW), index_map=lambda i: (0, i))],
            out_specs=[pl.BlockSpec((GW, FEAT), index_map=lambda i: (i, 0))],
            core_axis_name=("core", "subcore"),
            dimension_semantics=(pltpu.PARALLEL,),
        )(idx_hbm, out_hbm)

    return gk(table, idx)


def _att_body(seq_ref, W1_ref, f1w1_ref, f1b1_ref, f2w1_ref, f2b1_ref, bout1_ref,
              W2_ref, f1w2_ref, f1b2_ref, f2w2_ref, f2b2_ref, bout2_ref, out_ref):
    X = seq_ref[...].astype(jnp.bfloat16)
    rid = lax.broadcasted_iota(jnp.int32, (ROWS, ROWS), 0) // NB1
    cid = lax.broadcasted_iota(jnp.int32, (ROWS, ROWS), 1) // NB1
    mask = rid == cid

    vals = []
    for h in range(HEADS):
        S = jnp.dot(X, W1_ref[h].astype(jnp.bfloat16),
                    preferred_element_type=jnp.float32)
        Sb = S.astype(jnp.bfloat16)
        f1 = jnp.dot(Sb, f1w1_ref[h].astype(jnp.bfloat16),
                     preferred_element_type=jnp.float32) + f1b1_ref[h]
        f2 = jnp.dot(Sb, f2w1_ref[h].astype(jnp.bfloat16),
                     preferred_element_type=jnp.float32) + f2b1_ref[h]
        Z = _leaky(f1 + f2.reshape(1, ROWS))
        Z = jnp.where(mask, Z, -1e30)
        A = jax.nn.softmax(Z, axis=-1)
        V = jnp.dot(A.astype(jnp.bfloat16), Sb,
                    preferred_element_type=jnp.float32) + bout1_ref[h]
        vals.append(V.astype(jnp.bfloat16))
    h1 = jnp.concatenate(vals, axis=-1)

    rid8 = lax.broadcasted_iota(jnp.int32, (BB, ROWS), 0)
    cid8 = lax.broadcasted_iota(jnp.int32, (BB, ROWS), 1) // NB1
    mask8 = rid8 == cid8
    acc = jnp.zeros((BB, OUT), jnp.float32)
    for h in range(HEADS):
        S2 = jnp.dot(h1, W2_ref[h].astype(jnp.bfloat16),
                     preferred_element_type=jnp.float32)
        S2b = S2.astype(jnp.bfloat16)
        f1 = jnp.dot(S2b, f1w2_ref[h].astype(jnp.bfloat16),
                     preferred_element_type=jnp.float32) + f1b2_ref[h]
        f2 = jnp.dot(S2b, f2w2_ref[h].astype(jnp.bfloat16),
                     preferred_element_type=jnp.float32) + f2b2_ref[h]
        f1_sel = f1.reshape(BB, NB1)[:, 0:1]
        Z2 = _leaky(f1_sel + f2.reshape(1, ROWS))
        Z2 = jnp.where(mask8, Z2, -1e30)
        A2 = jax.nn.softmax(Z2, axis=-1)
        V2 = jnp.dot(A2.astype(jnp.bfloat16), S2b,
                     preferred_element_type=jnp.float32) + bout2_ref[h]
        acc = acc + V2
    out_ref[...] = acc * (1.0 / HEADS)


def _attention(seq, W1, f1w1, f1b1, f2w1, f2b1, bout1,
               W2, f1w2, f1b2, f2w2, f2b2, bout2):
    grid = BATCH // BB
    full = lambda *s: pl.BlockSpec(s, lambda i: tuple(0 for _ in s))
    return pl.pallas_call(
        _att_body,
        grid=(grid,),
        in_specs=[
            pl.BlockSpec((ROWS, FEAT), lambda i: (i, 0)),
            full(HEADS, FEAT, HID),
            full(HEADS, HID, 1),
            full(HEADS, 1),
            full(HEADS, HID, 1),
            full(HEADS, 1),
            full(HEADS, HID),
            full(HEADS, HEADS * HID, OUT),
            full(HEADS, OUT, 1),
            full(HEADS, 1),
            full(HEADS, OUT, 1),
            full(HEADS, 1),
            full(HEADS, OUT),
        ],
        out_specs=pl.BlockSpec((BB, OUT), lambda i: (i, 0)),
        out_shape=jax.ShapeDtypeStruct((BATCH, OUT), jnp.float32),
    )(seq, W1, f1w1, f1b1, f2w1, f2b1, bout1, W2, f1w2, f1b2, f2w2, f2b2, bout2)


def kernel(inputs, neighbors, table, W1, f1w1, f1b1, f2w1, f2b1, bout1,
           W2, f1w2, f1b2, f2w2, f2b2, bout2):
    idx = jnp.concatenate([inputs[:, None], neighbors], axis=1)
    idx = idx.reshape(1, BATCH * NB1).astype(jnp.int32)
    seq = _gather(table, idx)
    return _attention(seq, W1, f1w1, f1b1, f2w1, f2b1, bout1,
                      W2, f1w2, f1b2, f2w2, f2b2, bout2)


# speedup vs baseline: 1.5979x; 1.2716x over previous
import functools

import jax
import jax.numpy as jnp
from jax import lax
from jax.experimental import pallas as pl
from jax.experimental.pallas import tpu as pltpu
from jax.experimental.pallas import tpu_sc as plsc

BATCH = 2048
NB1 = 32
FEAT = 256
HID = 256
OUT = 128
HEADS = 2
BB = 8
ROWS = BB * NB1
GW = 128


def _leaky(x):
    return jnp.where(x >= 0, x, 0.2 * x)


def _gather(table, idx):
    n = idx.shape[1]
    mesh = plsc.VectorSubcoreMesh(core_axis_name="core", subcore_axis_name="subcore")

    @functools.partial(
        pl.kernel,
        out_type=jax.ShapeDtypeStruct((n, FEAT), jnp.float32),
        mesh=mesh,
    )
    def gk(table_hbm, idx_hbm, out_hbm):
        def body(i_vmem, o_vmem):
            pltpu.sync_copy(table_hbm.at[i_vmem.at[0]], o_vmem)

        pltpu.emit_pipeline(
            body,
            grid=(n // GW,),
            in_specs=[pl.BlockSpec((1, GW), index_map=lambda i: (0, i))],
            out_specs=[pl.BlockSpec((GW, FEAT), index_map=lambda i: (i, 0))],
            core_axis_name=("core", "subcore"),
            dimension_semantics=(pltpu.PARALLEL,),
        )(idx_hbm, out_hbm)

    return gk(table, idx)


def _att_body(seq_ref, W1_ref, Wf1_ref, fb1_ref, bout1_ref,
              W2_ref, Wf2_ref, fb2_ref, bout2_ref, out_ref):
    X = seq_ref[...].astype(jnp.bfloat16)
    rid = lax.broadcasted_iota(jnp.int32, (ROWS, ROWS), 0) // NB1
    cid = lax.broadcasted_iota(jnp.int32, (ROWS, ROWS), 1) // NB1
    maskb = (rid == cid).astype(jnp.bfloat16)

    Wf1 = Wf1_ref[...].astype(jnp.bfloat16)
    F1 = jnp.dot(X, Wf1, preferred_element_type=jnp.float32)
    G1 = lax.dot_general(Wf1, X, (((0,), (1,)), ((), ())),
                         preferred_element_type=jnp.float32)

    vals = []
    for h in range(HEADS):
        S = jnp.dot(X, W1_ref[h].astype(jnp.bfloat16),
                    preferred_element_type=jnp.float32)
        Sb = S.astype(jnp.bfloat16)
        f1 = F1[:, 2 * h:2 * h + 1] + fb1_ref[h:h + 1, 0:1]
        f2 = G1[2 * h + 1:2 * h + 2, :] + fb1_ref[h:h + 1, 1:2]
        Z = f1 + f2
        Z = jnp.maximum(Z, 0.2 * Z)
        E = jnp.exp(Z).astype(jnp.bfloat16) * maskb
        denom = jnp.sum(E.astype(jnp.float32), axis=-1, keepdims=True)
        recip = 1.0 / denom
        V = jnp.dot(E, Sb, preferred_element_type=jnp.float32) * recip + bout1_ref[h]
        vals.append(V.astype(jnp.bfloat16))
    h1 = jnp.concatenate(vals, axis=-1)

    rid8 = lax.broadcasted_iota(jnp.int32, (BB, ROWS), 0)
    cid8 = lax.broadcasted_iota(jnp.int32, (BB, ROWS), 1) // NB1
    mask8b = (rid8 == cid8).astype(jnp.bfloat16)

    Wf2 = Wf2_ref[...].astype(jnp.bfloat16)
    F2 = jnp.dot(h1, Wf2, preferred_element_type=jnp.float32)
    G2 = lax.dot_general(Wf2, h1, (((0,), (1,)), ((), ())),
                         preferred_element_type=jnp.float32)

    acc = jnp.zeros((BB, OUT), jnp.float32)
    for h in range(HEADS):
        S2 = jnp.dot(h1, W2_ref[h].astype(jnp.bfloat16),
                     preferred_element_type=jnp.float32)
        S2b = S2.astype(jnp.bfloat16)
        f1 = F2[:, 2 * h:2 * h + 1] + fb2_ref[h:h + 1, 0:1]
        f2 = G2[2 * h + 1:2 * h + 2, :] + fb2_ref[h:h + 1, 1:2]
        f1_sel = f1.reshape(BB, NB1)[:, 0:1]
        Z2 = f1_sel + f2
        Z2 = jnp.maximum(Z2, 0.2 * Z2)
        E2 = jnp.exp(Z2).astype(jnp.bfloat16) * mask8b
        denom2 = jnp.sum(E2.astype(jnp.float32), axis=-1, keepdims=True)
        recip2 = 1.0 / denom2
        V2 = jnp.dot(E2, S2b, preferred_element_type=jnp.float32) * recip2 + bout2_ref[h]
        acc = acc + V2
    out_ref[...] = acc * (1.0 / HEADS)


def _attention(seq, W1, Wf1, fb1, bout1, W2, Wf2, fb2, bout2):
    grid = BATCH // BB
    full = lambda *s: pl.BlockSpec(s, lambda i: tuple(0 for _ in s))
    return pl.pallas_call(
        _att_body,
        grid=(grid,),
        in_specs=[
            pl.BlockSpec((ROWS, FEAT), lambda i: (i, 0)),
            full(HEADS, FEAT, HID),
            full(FEAT, 2 * HEADS),
            full(HEADS, 2),
            full(HEADS, HID),
            full(HEADS, HEADS * HID, OUT),
            full(HEADS * HID, 2 * HEADS),
            full(HEADS, 2),
            full(HEADS, OUT),
        ],
        out_specs=pl.BlockSpec((BB, OUT), lambda i: (i, 0)),
        out_shape=jax.ShapeDtypeStruct((BATCH, OUT), jnp.float32),
    )(seq, W1, Wf1, fb1, bout1, W2, Wf2, fb2, bout2)


def kernel(inputs, neighbors, table, W1, f1w1, f1b1, f2w1, f2b1, bout1,
           W2, f1w2, f1b2, f2w2, f2b2, bout2):
    idx = jnp.concatenate([inputs[:, None], neighbors], axis=1)
    idx = idx.reshape(1, BATCH * NB1).astype(jnp.int32)
    seq = _gather(table, idx)
    Wf1 = jnp.stack([jnp.einsum('do,ok->dk', W1[h], w)[:, 0]
                     for h in range(HEADS) for w in (f1w1[h], f2w1[h])], axis=-1)
    Wf2 = jnp.stack([jnp.einsum('do,ok->dk', W2[h], w)[:, 0]
                     for h in range(HEADS) for w in (f1w2[h], f2w2[h])], axis=-1)
    fb1 = jnp.concatenate([f1b1, f2b1], axis=-1)
    fb2 = jnp.concatenate([f1b2, f2b2], axis=-1)
    return _attention(seq, W1, Wf1, fb1, bout1, W2, Wf2, fb2, bout2)


# speedup vs baseline: 2.7307x; 1.7089x over previous
import functools

import jax
import jax.numpy as jnp
from jax import lax
from jax.experimental import pallas as pl
from jax.experimental.pallas import tpu as pltpu
from jax.experimental.pallas import tpu_sc as plsc

BATCH = 2048
NB1 = 32
FEAT = 256
HID = 256
OUT = 128
HEADS = 2
BB = 8
ROWS = BB * NB1
GW = 128


def _leaky(x):
    return jnp.where(x >= 0, x, 0.2 * x)


def _gather(table, idx):
    n = idx.shape[1]
    mesh = plsc.VectorSubcoreMesh(core_axis_name="core", subcore_axis_name="subcore")

    @functools.partial(
        pl.kernel,
        out_type=jax.ShapeDtypeStruct((n, FEAT), jnp.float32),
        mesh=mesh,
    )
    def gk(table_hbm, idx_hbm, out_hbm):
        def body(i_vmem, o_vmem):
            pltpu.sync_copy(table_hbm.at[i_vmem.at[0]], o_vmem)

        pltpu.emit_pipeline(
            body,
            grid=(n // GW,),
            in_specs=[pl.BlockSpec((1, GW), index_map=lambda i: (0, i))],
            out_specs=[pl.BlockSpec((GW, FEAT), index_map=lambda i: (i, 0))],
            core_axis_name=("core", "subcore"),
            dimension_semantics=(pltpu.PARALLEL,),
        )(idx_hbm, out_hbm)

    return gk(table, idx)


def _att_body(seq_ref, W1_ref, Wf1_ref, fb1_ref, bout1_ref,
              W2_ref, Wf2_ref, fb2_ref, bout2_ref, out_ref):
    X = seq_ref[...].astype(jnp.bfloat16)
    rid = lax.broadcasted_iota(jnp.int32, (ROWS, ROWS), 0) // NB1
    cid = lax.broadcasted_iota(jnp.int32, (ROWS, ROWS), 1) // NB1
    maskb = (rid == cid).astype(jnp.bfloat16)

    Wf1 = Wf1_ref[...].astype(jnp.bfloat16)
    F1 = jnp.dot(X, Wf1, preferred_element_type=jnp.float32)
    G1 = lax.dot_general(Wf1, X, (((0,), (1,)), ((), ())),
                         preferred_element_type=jnp.float32)

    vals = []
    for h in range(HEADS):
        S = jnp.dot(X, W1_ref[h].astype(jnp.bfloat16),
                    preferred_element_type=jnp.float32)
        Sb = S.astype(jnp.bfloat16)
        f1 = F1[:, 2 * h:2 * h + 1] + fb1_ref[h:h + 1, 0:1]
        f2 = G1[2 * h + 1:2 * h + 2, :] + fb1_ref[h:h + 1, 1:2]
        Z = f1 + f2
        Z = jnp.maximum(Z, 0.2 * Z)
        E = jnp.exp(Z).astype(jnp.bfloat16) * maskb
        denom = jnp.sum(E.astype(jnp.float32), axis=-1, keepdims=True)
        recip = 1.0 / denom
        V = jnp.dot(E, Sb, preferred_element_type=jnp.float32) * recip + bout1_ref[h]
        vals.append(V.astype(jnp.bfloat16))
    h1 = jnp.concatenate(vals, axis=-1)

    Wf2 = Wf2_ref[...].astype(jnp.bfloat16)
    F2 = jnp.dot(h1, Wf2, preferred_element_type=jnp.float32)
    G2 = lax.dot_general(Wf2, h1, (((0,), (1,)), ((), ())),
                         preferred_element_type=jnp.float32)

    acc = jnp.zeros((ROWS, OUT), jnp.float32)
    for h in range(HEADS):
        S2 = jnp.dot(h1, W2_ref[h].astype(jnp.bfloat16),
                     preferred_element_type=jnp.float32)
        S2b = S2.astype(jnp.bfloat16)
        f1 = F2[:, 2 * h:2 * h + 1] + fb2_ref[h:h + 1, 0:1]
        f2 = G2[2 * h + 1:2 * h + 2, :] + fb2_ref[h:h + 1, 1:2]
        Z2 = f1 + f2
        Z2 = jnp.maximum(Z2, 0.2 * Z2)
        E2 = jnp.exp(Z2).astype(jnp.bfloat16) * maskb
        denom2 = jnp.sum(E2.astype(jnp.float32), axis=-1, keepdims=True)
        recip2 = 1.0 / denom2
        V2 = jnp.dot(E2, S2b, preferred_element_type=jnp.float32) * recip2 + bout2_ref[h]
        acc = acc + V2
    out_ref[...] = acc.reshape(BB, NB1, OUT)[:, 0, :] * (1.0 / HEADS)


def _attention(seq, W1, Wf1, fb1, bout1, W2, Wf2, fb2, bout2):
    grid = BATCH // BB
    full = lambda *s: pl.BlockSpec(s, lambda i: tuple(0 for _ in s))
    return pl.pallas_call(
        _att_body,
        grid=(grid,),
        in_specs=[
            pl.BlockSpec((ROWS, FEAT), lambda i: (i, 0)),
            full(HEADS, FEAT, HID),
            full(FEAT, 2 * HEADS),
            full(HEADS, 2),
            full(HEADS, HID),
            full(HEADS, HEADS * HID, OUT),
            full(HEADS * HID, 2 * HEADS),
            full(HEADS, 2),
            full(HEADS, OUT),
        ],
        out_specs=pl.BlockSpec((BB, OUT), lambda i: (i, 0)),
        out_shape=jax.ShapeDtypeStruct((BATCH, OUT), jnp.float32),
    )(seq, W1, Wf1, fb1, bout1, W2, Wf2, fb2, bout2)


def kernel(inputs, neighbors, table, W1, f1w1, f1b1, f2w1, f2b1, bout1,
           W2, f1w2, f1b2, f2w2, f2b2, bout2):
    idx = jnp.concatenate([inputs[:, None], neighbors], axis=1)
    idx = idx.reshape(1, BATCH * NB1).astype(jnp.int32)
    seq = _gather(table, idx)
    Wf1 = jnp.stack([jnp.einsum('do,ok->dk', W1[h], w)[:, 0]
                     for h in range(HEADS) for w in (f1w1[h], f2w1[h])], axis=-1)
    Wf2 = jnp.stack([jnp.einsum('do,ok->dk', W2[h], w)[:, 0]
                     for h in range(HEADS) for w in (f1w2[h], f2w2[h])], axis=-1)
    fb1 = jnp.concatenate([f1b1, f2b1], axis=-1)
    fb2 = jnp.concatenate([f1b2, f2b2], axis=-1)
    return _attention(seq, W1, Wf1, fb1, bout1, W2, Wf2, fb2, bout2)
